# baseline (speedup 1.0000x reference)
import functools

import jax
import jax.numpy as jnp
from jax import lax
from jax.experimental import pallas as pl
from jax.experimental.pallas import tpu as pltpu
from jax.experimental.pallas import tpu_sc as plsc

_NUM_CORES = 2
_NUM_SUBCORES = 16
_NW = _NUM_CORES * _NUM_SUBCORES
_LANES = 16
_BLK = 16
_TILE = 128


@functools.lru_cache(maxsize=None)
def _make_sc_bottom(T):
    ntile = T // _TILE
    nblk_total = T // _BLK
    mesh = plsc.VectorSubcoreMesh(core_axis_name="c", subcore_axis_name="s")

    @functools.partial(
        pl.kernel,
        out_type=jax.ShapeDtypeStruct((_NW, 2 * T), jnp.float32),
        mesh=mesh,
        scratch_types=[
            pltpu.VMEM((2 * _BLK, T), jnp.float32),
            pltpu.VMEM((2 * T,), jnp.float32),
            [pltpu.SemaphoreType.DMA] * 2,
        ],
        compiler_params=pltpu.CompilerParams(needs_layout_passes=False),
    )
    def sc_bottom(x_hbm, part_hbm, buf, acc, sems):
        cid = lax.axis_index("c")
        sid = lax.axis_index("s")
        wid = cid * _NUM_SUBCORES + sid

        zero16 = jnp.zeros((_LANES,), jnp.float32)

        @plsc.parallel_loop(0, (2 * T) // _LANES, unroll=4)
        def _(k):
            acc[pl.ds(k * _LANES, _LANES)] = zero16

        half = nblk_total // 2
        blocks = [half + wid, nblk_total - 1 - wid]

        iota = lax.iota(jnp.int32, _LANES)

        def start_blk_dma(idx, slot):
            blk_id = blocks[idx]
            return pltpu.async_copy(
                x_hbm.at[pl.ds(blk_id * _BLK, _BLK), :],
                buf.at[pl.ds(slot * _BLK, _BLK), :],
                sems[slot],
            )

        copies = [None] * len(blocks)
        copies[0] = start_blk_dma(0, 0)
        for bi in range(len(blocks)):
            if bi + 1 < len(blocks):
                copies[bi + 1] = start_blk_dma(bi + 1, (bi + 1) % 2)
            copies[bi].wait()
            slot0 = (bi % 2) * _BLK
            row0 = blocks[bi] * _BLK

            def row_body(rr, carry):
                r = slot0 + rr
                i = row0 + rr
                shift = T - i
                lo = jnp.maximum(0, 2 * i - T + 2)
                t0 = jnp.minimum(lo // _TILE, ntile - 1)

                jb = t0 * _TILE
                for c in range(_TILE // _LANES):
                    jvec = iota + (jb + c * _LANES)
                    xv = buf[r, pl.ds(jb + c * _LANES, _LANES)]
                    plsc.addupdate_scatter(
                        acc, [jvec + shift], xv,
                        mask=(jvec >= lo) & (jvec < T - 1),
                    )

                @plsc.parallel_loop(t0 + 1, ntile - 1)
                def _(t):
                    tb = t * _TILE
                    for c in range(_TILE // _LANES):
                        xv = buf[r, pl.ds(tb + c * _LANES, _LANES)]
                        plsc.addupdate(
                            acc.at[pl.ds(tb + c * _LANES + shift, _LANES)], xv
                        )

                @pl.when(t0 < ntile - 1)
                def _():
                    tb = (ntile - 1) * _TILE
                    for c in range(_TILE // _LANES - 1):
                        xv = buf[r, pl.ds(tb + c * _LANES, _LANES)]
                        plsc.addupdate(
                            acc.at[pl.ds(tb + c * _LANES + shift, _LANES)], xv
                        )
                    cb = tb + _TILE - _LANES
                    jvec = iota + cb
                    xv = buf[r, pl.ds(cb, _LANES)]
                    plsc.addupdate_scatter(
                        acc, [jvec + shift], xv, mask=jvec < T - 1
                    )
                return carry

            lax.fori_loop(0, _BLK, row_body, 0)

        pltpu.sync_copy(acc, part_hbm.at[wid])

    return sc_bottom


def _tc_top_body(T, S, nslab, x_ref, sc_ref, out_ref, acc_ref):
    s = pl.program_id(0)
    W = T + S

    @pl.when(s == 0)
    def _():
        acc_ref[...] = jnp.zeros_like(acc_ref)

    x = x_ref[...]
    col = lax.broadcasted_iota(jnp.int32, (S, T), 1)
    x = jnp.where(col < T - 1, x, 0.0)
    y = jnp.concatenate([x, jnp.zeros((S, W - T), jnp.float32)], axis=1)
    rows, k = S, 1
    while rows > 1:
        rolled = pltpu.roll(y, k, 1)
        ev = rolled.reshape(rows // 2, 2, W)[:, 0, :]
        od = y.reshape(rows // 2, 2, W)[:, 1, :]
        y = ev + od
        rows //= 2
        k *= 2
    colsum = y
    for sv in range(nslab):
        @pl.when(s == sv)
        def _():
            base = T - (S - 1) - sv * S
            acc_ref[0, base:base + W] += colsum[0, :]

    @pl.when(s == nslab - 1)
    def _():
        tot = jnp.sum(sc_ref[...], axis=0, keepdims=True) + acc_ref[0:1, : 2 * T]
        b = lax.broadcasted_iota(jnp.int32, (1, 2 * T), 1)
        cnt = jnp.where(b < T, 2 * b - T - 1, 2 * T - 1 - b)
        cnt = jnp.maximum(cnt, 0).astype(jnp.float32)
        out_ref[...] = tot / cnt


@functools.lru_cache(maxsize=None)
def _make_tc_top(T):
    S = 128
    nslab = (T // 2) // S
    return pl.pallas_call(
        functools.partial(_tc_top_body, T, S, nslab),
        grid=(nslab,),
        in_specs=[
            pl.BlockSpec((S, T), lambda s: (s, 0)),
            pl.BlockSpec((_NW, 2 * T), lambda s: (0, 0)),
        ],
        out_specs=pl.BlockSpec((1, 2 * T), lambda s: (0, 0)),
        out_shape=jax.ShapeDtypeStruct((1, 2 * T), jnp.float32),
        scratch_shapes=[pltpu.VMEM((1, 2 * T + S), jnp.float32)],
    )


def kernel(inputs):
    T = inputs.shape[0]
    sc_part = _make_sc_bottom(T)(inputs)
    out = _make_tc_top(T)(inputs, sc_part)
    return out.reshape(2 * T)


# speedup vs baseline: 1.3341x; 1.3341x over previous
import functools

import jax
import jax.numpy as jnp
from jax import lax
from jax.experimental import pallas as pl
from jax.experimental.pallas import tpu as pltpu
from jax.experimental.pallas import tpu_sc as plsc

_NUM_CORES = 2
_NUM_SUBCORES = 16
_NW = _NUM_CORES * _NUM_SUBCORES
_LANES = 16
_BLK = 16
_TILE = 128


@functools.lru_cache(maxsize=None)
def _make_sc_bottom(T):
    ntile = T // _TILE
    nblk_total = T // _BLK
    mesh = plsc.VectorSubcoreMesh(core_axis_name="c", subcore_axis_name="s")

    @functools.partial(
        pl.kernel,
        out_type=jax.ShapeDtypeStruct((_NW, 2 * T), jnp.float32),
        mesh=mesh,
        scratch_types=[
            pltpu.VMEM((2 * _BLK, T), jnp.float32),
            pltpu.VMEM((2 * T,), jnp.float32),
            [pltpu.SemaphoreType.DMA] * 2,
        ],
        compiler_params=pltpu.CompilerParams(needs_layout_passes=False),
    )
    def sc_bottom(x_hbm, part_hbm, buf, acc, sems):
        cid = lax.axis_index("c")
        sid = lax.axis_index("s")
        wid = cid * _NUM_SUBCORES + sid

        zero16 = jnp.zeros((_LANES,), jnp.float32)

        @plsc.parallel_loop(0, (2 * T) // _LANES, unroll=4)
        def _(k):
            acc[pl.ds(k * _LANES, _LANES)] = zero16

        half = nblk_total // 2
        blocks = [half + wid, nblk_total - 1 - wid]

        iota = lax.iota(jnp.int32, _LANES)

        def start_blk_dma(idx, slot):
            blk_id = blocks[idx]
            return pltpu.async_copy(
                x_hbm.at[pl.ds(blk_id * _BLK, _BLK), :],
                buf.at[pl.ds(slot * _BLK, _BLK), :],
                sems[slot],
            )

        copies = [None] * len(blocks)
        copies[0] = start_blk_dma(0, 0)
        for bi in range(len(blocks)):
            if bi + 1 < len(blocks):
                copies[bi + 1] = start_blk_dma(bi + 1, (bi + 1) % 2)
            copies[bi].wait()
            slot0 = (bi % 2) * _BLK
            row0 = blocks[bi] * _BLK

            def row_body(rr, carry):
                r = slot0 + rr
                i = row0 + rr
                shift = T - i
                lo = jnp.maximum(0, 2 * i - T + 2)
                t0 = jnp.minimum(lo // _TILE, ntile - 1)

                jb = t0 * _TILE
                for c in range(_TILE // _LANES):
                    jvec = iota + (jb + c * _LANES)
                    xv = buf[r, pl.ds(jb + c * _LANES, _LANES)]
                    plsc.addupdate_scatter(
                        acc, [jvec + shift], xv,
                        mask=(jvec >= lo) & (jvec < T - 1),
                    )

                @plsc.parallel_loop(t0 + 1, ntile - 1)
                def _(t):
                    tb = t * _TILE
                    for c in range(_TILE // _LANES):
                        xv = buf[r, pl.ds(tb + c * _LANES, _LANES)]
                        plsc.addupdate(
                            acc.at[pl.ds(tb + c * _LANES + shift, _LANES)], xv
                        )

                @pl.when(t0 < ntile - 1)
                def _():
                    tb = (ntile - 1) * _TILE
                    for c in range(_TILE // _LANES - 1):
                        xv = buf[r, pl.ds(tb + c * _LANES, _LANES)]
                        plsc.addupdate(
                            acc.at[pl.ds(tb + c * _LANES + shift, _LANES)], xv
                        )
                    cb = tb + _TILE - _LANES
                    jvec = iota + cb
                    xv = buf[r, pl.ds(cb, _LANES)]
                    plsc.addupdate_scatter(
                        acc, [jvec + shift], xv, mask=jvec < T - 1
                    )
                return carry

            lax.fori_loop(0, _BLK, row_body, 0)

        pltpu.sync_copy(acc, part_hbm.at[wid])

    return sc_bottom


def _tc_top_body(T, S, nslab, x_ref, out_ref):
    s = pl.program_id(0)
    W = T + S

    @pl.when(s == 0)
    def _():
        out_ref[...] = jnp.zeros_like(out_ref)

    x = x_ref[...]
    col = lax.broadcasted_iota(jnp.int32, (S, T), 1)
    x = jnp.where(col < T - 1, x, 0.0)
    y = jnp.concatenate([x, jnp.zeros((S, W - T), jnp.float32)], axis=1)
    rows, k = S, 1
    while rows > 1:
        rolled = pltpu.roll(y, k, 1)
        ev = rolled.reshape(rows // 2, 2, W)[:, 0, :]
        od = y.reshape(rows // 2, 2, W)[:, 1, :]
        y = ev + od
        rows //= 2
        k *= 2
    colsum = y
    for sv in range(nslab):
        @pl.when(s == sv)
        def _():
            base = T - (S - 1) - sv * S
            out_ref[0, base:base + W] += colsum[0, :]


@functools.lru_cache(maxsize=None)
def _make_tc_top(T):
    S = 128
    nslab = (T // 2) // S
    return pl.pallas_call(
        functools.partial(_tc_top_body, T, S, nslab),
        grid=(nslab,),
        in_specs=[pl.BlockSpec((S, T), lambda s: (s, 0))],
        out_specs=pl.BlockSpec((1, 2 * T + S), lambda s: (0, 0)),
        out_shape=jax.ShapeDtypeStruct((1, 2 * T + S), jnp.float32),
    )


def _combine_body(T, p_ref, t_ref, o_ref):
    s = jnp.sum(p_ref[...], axis=0, keepdims=True) + t_ref[0:1, : 2 * T]
    b = lax.broadcasted_iota(jnp.int32, (1, 2 * T), 1)
    cnt = jnp.where(b < T, 2 * b - T - 1, 2 * T - 1 - b)
    cnt = jnp.maximum(cnt, 0).astype(jnp.float32)
    o_ref[...] = s / cnt


def kernel(inputs):
    T = inputs.shape[0]
    sc_part = _make_sc_bottom(T)(inputs)
    tc_part = _make_tc_top(T)(inputs)
    out = pl.pallas_call(
        functools.partial(_combine_body, T),
        out_shape=jax.ShapeDtypeStruct((1, 2 * T), jnp.float32),
    )(sc_part, tc_part)
    return out.reshape(2 * T)
